# Initial kernel scaffold; baseline (speedup 1.0000x reference)
import jax
import jax.numpy as jnp
from jax.experimental import pallas as pl


def kernel(feat):
    raise NotImplementedError("write your pallas kernel here")


# speedup vs baseline: 19.3349x; 19.3349x over previous
import functools

import jax
import jax.numpy as jnp
import numpy as np
from jax import lax
from jax.experimental import pallas as pl

_NUM_ANGLE = 100
_NUM_RHO = 100
_RHO_PAD = 128
_ANGLE_BLK = 2


def _bin_table(H, W, numangle, numrho):
    irho = float(int(np.sqrt(H * H + W * W) + 1)) / float(numrho - 1)
    itheta = np.pi / numangle
    angles = np.arange(numangle, dtype=np.float64) * itheta
    tab_cos = (np.cos(angles) / irho).astype(np.float32)
    tab_sin = (np.sin(angles) / irho).astype(np.float32)
    ys, xs = np.meshgrid(np.arange(H), np.arange(W), indexing="ij")
    xx = (xs - (W // 2)).reshape(-1).astype(np.float32)
    yy = (ys - (H // 2)).reshape(-1).astype(np.float32)
    proj = xx[None, :] * tab_cos[:, None] + yy[None, :] * tab_sin[:, None]
    proj = proj.astype(np.float32)
    r = np.where(proj >= 0, np.floor(proj + 0.5), np.ceil(proj - 0.5))
    r = r.astype(np.int32) + (numrho // 2)
    return np.clip(r, 0, numrho - 1)


def _hough_body(r_ref, feat_ref, out_ref, *, pp):
    i128 = lax.broadcasted_iota(jnp.int32, (pp, _RHO_PAD), 1)
    oh0 = (r_ref[0, 0, :][:, None] == i128).astype(jnp.bfloat16)
    oh1 = (r_ref[1, 0, :][:, None] == i128).astype(jnp.bfloat16)
    oh = jnp.concatenate([oh0, oh1], axis=1)
    out_ref[0] = jax.lax.dot_general(
        feat_ref[...], oh,
        dimension_numbers=(((1,), (0,)), ((), ())),
        preferred_element_type=jnp.float32,
    )


def kernel(feat):
    N, C, H, W = feat.shape
    NC = N * C
    P = H * W
    PP = ((P + 1023) // 1024) * 1024
    A, R = _NUM_ANGLE, _NUM_RHO

    r_np = _bin_table(H, W, A, R)
    r_pad = np.full((A, 1, PP), R, dtype=np.int32)
    r_pad[:, 0, :P] = r_np
    r_tab = jnp.asarray(r_pad)

    feat2 = feat.reshape(NC, P).astype(jnp.bfloat16)
    feat2 = jnp.pad(feat2, ((0, 0), (0, PP - P)))

    grid = (A // _ANGLE_BLK,)
    out = pl.pallas_call(
        functools.partial(_hough_body, pp=PP),
        grid=grid,
        in_specs=[
            pl.BlockSpec((_ANGLE_BLK, 1, PP), lambda a: (a, 0, 0)),
            pl.BlockSpec((NC, PP), lambda a: (0, 0)),
        ],
        out_specs=pl.BlockSpec((1, NC, _ANGLE_BLK * _RHO_PAD), lambda a: (a, 0, 0)),
        out_shape=jax.ShapeDtypeStruct((A // _ANGLE_BLK, NC, _ANGLE_BLK * _RHO_PAD), jnp.float32),
    )(r_tab, feat2)

    out = out.reshape(A // _ANGLE_BLK, NC, _ANGLE_BLK, _RHO_PAD)[:, :, :, :R]
    out = out.transpose(1, 0, 2, 3).reshape(N, C, A, R)
    return out
